# Initial kernel scaffold; baseline (speedup 1.0000x reference)
import jax
import jax.numpy as jnp
from jax.experimental import pallas as pl


def kernel(audio, depth, rate):
    raise NotImplementedError("write your pallas kernel here")


# speedup vs baseline: 6063.6434x; 6063.6434x over previous
import jax
import jax.numpy as jnp
from jax import lax
from jax.experimental import pallas as pl
from jax.experimental.pallas import tpu as pltpu

_SR = 44100
_N = 44100
_MAXD = 220
_K = 8
_ROWS = 352
_LANES = 128
_OUT_PAD = 256


def _vibrato_body(depth_ref, rate_ref, audio_ref, out_ref):
    depth = depth_ref[0]
    rate = rate_ref[0]
    row = lax.broadcasted_iota(jnp.int32, (_ROWS, _LANES), 0)
    col = lax.broadcasted_iota(jnp.int32, (_ROWS, _LANES), 1)
    n = row * _LANES + col
    t = n.astype(jnp.float32) / float(_SR)
    lfo = 0.5 * (1.0 + jnp.sin(2.0 * jnp.pi * rate * t))
    x = depth * lfo
    q = x.astype(jnp.int32)
    valid = n < _N
    acc = jnp.zeros((8, _OUT_PAD), jnp.float32)
    for k in range(1, _K):
        cnt = jnp.sum(jnp.where(valid & (q == k), 1.0, 0.0))
        seg = audio_ref[:, :k]
        shifted = jnp.pad(seg, ((0, 0), (_MAXD - k, _OUT_PAD - _MAXD)))
        acc = acc + cnt * shifted
    out_ref[:, :] = acc * (1.0 / float(_N))


@jax.jit
def kernel(audio, depth, rate):
    B, C, N = audio.shape
    flat = audio.reshape(B * C, N)
    out = pl.pallas_call(
        _vibrato_body,
        grid=(1,),
        in_specs=[
            pl.BlockSpec(memory_space=pltpu.SMEM),
            pl.BlockSpec(memory_space=pltpu.SMEM),
            pl.BlockSpec((8, _LANES), lambda i: (0, 0)),
        ],
        out_specs=pl.BlockSpec((8, _OUT_PAD), lambda i: (0, 0)),
        out_shape=jax.ShapeDtypeStruct((8, _OUT_PAD), jnp.float32),
    )(depth.reshape(1), rate.reshape(1), flat)
    return out[:, :_MAXD].reshape(B, C, _MAXD)
